# baseline (speedup 1.0000x reference)
import functools

import jax
import jax.numpy as jnp
from jax import lax
from jax.experimental import pallas as pl
from jax.experimental.pallas import tpu as pltpu
from jax.experimental.pallas import tpu_sc as plsc

_N = 100000
_E = 3200000
_D = 16
_ITERS = 60
_NC = 2
_NS = 16
_NP = 100352
_NT = _NP // _NS
_EH = _E // _NC
_ET = _EH // _NS
_C = 2000
_NCH_E = _ET // _C
_NCH_Z = _N // _C
_VPC = _C // 16
_FQ = _NT // 4


def _encoder_body(feats128_ref, m_ref, b_ref, ei_ref,
                  rew_ref, exw_ref, srco_ref, dsto_ref):
    x = jnp.dot(feats128_ref[...], m_ref[...],
                preferred_element_type=jnp.float32,
                precision=lax.Precision.HIGHEST)
    x = x + b_ref[0, 0]
    r = -jax.nn.softplus(x)
    n = r.shape[0] * r.shape[1]
    rew_ref[...] = r.reshape(n)
    exw_ref[...] = jnp.exp(r).reshape(n)
    srco_ref[...] = ei_ref[0]
    dsto_ref[...] = ei_ref[1]


def _log_body(p_ref, out_ref):
    out_ref[...] = jnp.log(p_ref[0] + p_ref[1])


def _combine_into_zrep(pin_hbm, zrep, bufLA, bufHA, bufLB, bufHB,
                       semA, semB):
    def _issue(off, bl, bh, sem):
        pltpu.async_copy(pin_hbm.at[pl.ds(off, _C)], bl, sem)
        pltpu.async_copy(pin_hbm.at[pl.ds(_NP + off, _C)], bh, sem)

    def _wait(off, bl, bh, sem):
        pltpu.make_async_copy(pin_hbm.at[pl.ds(off, _C)], bl, sem).wait()
        pltpu.make_async_copy(pin_hbm.at[pl.ds(_NP + off, _C)], bh, sem).wait()

    def _compute(zoff, bl, bh):
        def _m(i, c):
            s = pl.ds(i * 16, 16)
            zrep[pl.ds(zoff + i * 16, 16)] = bl[s] + bh[s]
            return c

        lax.fori_loop(0, _VPC, _m, 0)

    _issue(0, bufLA, bufHA, semA)

    def _j(j, c):
        kA = 2 * j
        kB = 2 * j + 1
        _wait(kA * _C, bufLA, bufHA, semA)
        _issue(kB * _C, bufLB, bufHB, semB)
        _compute(kA * _C, bufLA, bufHA)
        _wait(kB * _C, bufLB, bufHB, semB)

        @pl.when(kB + 1 < _NCH_Z)
        def _():
            _issue((kB + 1) * _C, bufLA, bufHA, semA)

        _compute(kB * _C, bufLB, bufHB)
        return c

    lax.fori_loop(0, _NCH_Z // 2, _j, 0)


def _fp_iter_body(src_hbm, dst_hbm, w_hbm, sinkf_hbm, pin_hbm,
                  pout_hbm,
                  srcA, dstA, wA, tvA,
                  srcB, dstB, wB, tvB,
                  zrep,
                  semz, semlA, semlB, semsA, semsB,
                  zacc):
    cid = lax.axis_index("c")
    tid = lax.axis_index("s")
    ebase = cid * _EH + tid * _ET
    nsl = tid * _NT

    def _lin_issue(off, sv, dv, wvv, sem):
        pltpu.async_copy(src_hbm.at[pl.ds(off, _C)], sv, sem)
        pltpu.async_copy(dst_hbm.at[pl.ds(off, _C)], dv, sem)
        pltpu.async_copy(w_hbm.at[pl.ds(off, _C)], wvv, sem)

    def _lin_wait(off, sv, dv, wvv, sem):
        pltpu.make_async_copy(src_hbm.at[pl.ds(off, _C)], sv, sem).wait()
        pltpu.make_async_copy(dst_hbm.at[pl.ds(off, _C)], dv, sem).wait()
        pltpu.make_async_copy(w_hbm.at[pl.ds(off, _C)], wvv, sem).wait()

    def _gather_mul(tvv, wvv, dvv):
        def _m(i, c2):
            s = pl.ds(i * 16, 16)
            zd = plsc.load_gather(zrep, [dvv[s]])
            tvv[s] = wvv[s] * zd
            return c2

        lax.fori_loop(0, _VPC, _m, 0)

    _combine_into_zrep(pin_hbm, zrep, wA, tvA, wB, tvB, semlA, semlB)

    def _zero_tvA(i, c):
        tvA[pl.ds(i * 16, 16)] = jnp.zeros((16,), jnp.float32)
        return c

    lax.fori_loop(0, _FQ // 16, _zero_tvA, 0)

    def _zq(q, c):
        pltpu.sync_copy(tvA.at[pl.ds(0, _FQ)],
                        zacc.at[pl.ds(nsl + q * _FQ, _FQ)])
        return c

    lax.fori_loop(0, 4, _zq, 0)
    plsc.subcore_barrier()

    _lin_issue(ebase, srcA, dstA, wA, semlA)

    def _chunk2(j, cc):
        kA = 2 * j
        kB = 2 * j + 1
        offA = ebase + kA * _C
        offB = ebase + kB * _C
        _lin_wait(offA, srcA, dstA, wA, semlA)

        @pl.when(j > 0)
        def _():
            pltpu.make_async_copy(tvB, zacc.at[srcB], semsB).wait()

        _lin_issue(offB, srcB, dstB, wB, semlB)
        _gather_mul(tvA, wA, dstA)
        dA = pltpu.async_copy(tvA, zacc.at[srcA], semsA, add=True)
        _lin_wait(offB, srcB, dstB, wB, semlB)
        dA.wait()

        @pl.when(kB + 1 < _NCH_E)
        def _():
            _lin_issue(ebase + (kB + 1) * _C, srcA, dstA, wA, semlA)

        _gather_mul(tvB, wB, dstB)
        pltpu.async_copy(tvB, zacc.at[srcB], semsB, add=True)
        return cc

    lax.fori_loop(0, _NCH_E // 2, _chunk2, 0)
    pltpu.make_async_copy(tvB, zacc.at[srcB], semsB).wait()
    plsc.subcore_barrier()

    def _finq(q, c):
        accsl = pl.ds(nsl + q * _FQ, _FQ)
        outsl = pl.ds(cid * _NP + nsl + q * _FQ, _FQ)
        sub = pl.ds(0, _FQ)
        pltpu.sync_copy(zacc.at[accsl], tvA.at[sub])
        pltpu.sync_copy(sinkf_hbm.at[pl.ds(nsl + q * _FQ, _FQ)],
                        wB.at[sub])

        def _fin(i, c2):
            s = pl.ds(i * 16, 16)
            sf = wB[s]
            tvA[s] = sf * 0.5 + (1.0 - sf) * tvA[s]
            return c2

        lax.fori_loop(0, _FQ // 16, _fin, 0)
        pltpu.sync_copy(tvA.at[sub], pout_hbm.at[outsl])
        return c

    lax.fori_loop(0, 4, _finq, 0)


def _probs_body(src_hbm, dst_hbm, w_hbm, pin_hbm,
                probs_hbm,
                srcA, dstA, wA, tvA,
                srcB, dstB, wB, tvB,
                zrep,
                semz, semlA, semlB, semsA, semsB):
    cid = lax.axis_index("c")
    tid = lax.axis_index("s")
    ebase = cid * _EH + tid * _ET

    _combine_into_zrep(pin_hbm, zrep, wA, tvA, wB, tvB, semlA, semlB)

    def _lin_issue(off, sv, dv, wvv, sem):
        pltpu.async_copy(src_hbm.at[pl.ds(off, _C)], sv, sem)
        pltpu.async_copy(dst_hbm.at[pl.ds(off, _C)], dv, sem)
        pltpu.async_copy(w_hbm.at[pl.ds(off, _C)], wvv, sem)

    def _lin_wait(off, sv, dv, wvv, sem):
        pltpu.make_async_copy(src_hbm.at[pl.ds(off, _C)], sv, sem).wait()
        pltpu.make_async_copy(dst_hbm.at[pl.ds(off, _C)], dv, sem).wait()
        pltpu.make_async_copy(w_hbm.at[pl.ds(off, _C)], wvv, sem).wait()

    def _probs_compute(tvv, wvv, svv, dvv):
        def _p(i, c2):
            s = pl.ds(i * 16, 16)
            zd = plsc.load_gather(zrep, [dvv[s]])
            zs = plsc.load_gather(zrep, [svv[s]])
            tvv[s] = wvv[s] * zd / zs
            return c2

        lax.fori_loop(0, _VPC, _p, 0)

    _lin_issue(ebase, srcA, dstA, wA, semlA)

    def _chunk2(j, cc):
        kA = 2 * j
        kB = 2 * j + 1
        offA = ebase + kA * _C
        offB = ebase + kB * _C
        _lin_wait(offA, srcA, dstA, wA, semlA)
        _lin_issue(offB, srcB, dstB, wB, semlB)
        _probs_compute(tvA, wA, srcA, dstA)
        pltpu.sync_copy(tvA, probs_hbm.at[pl.ds(offA, _C)])
        _lin_wait(offB, srcB, dstB, wB, semlB)

        @pl.when(kB + 1 < _NCH_E)
        def _():
            _lin_issue(ebase + (kB + 1) * _C, srcA, dstA, wA, semlA)

        _probs_compute(tvB, wB, srcB, dstB)
        pltpu.sync_copy(tvB, probs_hbm.at[pl.ds(offB, _C)])
        return cc

    lax.fori_loop(0, _NCH_E // 2, _chunk2, 0)


_SC_SCRATCH = [
    pltpu.VMEM((_C,), jnp.int32),
    pltpu.VMEM((_C,), jnp.int32),
    pltpu.VMEM((_C,), jnp.float32),
    pltpu.VMEM((_C,), jnp.float32),
    pltpu.VMEM((_C,), jnp.int32),
    pltpu.VMEM((_C,), jnp.int32),
    pltpu.VMEM((_C,), jnp.float32),
    pltpu.VMEM((_C,), jnp.float32),
    pltpu.VMEM((_N,), jnp.float32),
    pltpu.SemaphoreType.DMA,
    pltpu.SemaphoreType.DMA,
    pltpu.SemaphoreType.DMA,
    pltpu.SemaphoreType.DMA,
    pltpu.SemaphoreType.DMA,
]


def kernel(edge_index, edge_feats, sink_node_mask, W, b):
    sinkf = jnp.pad(sink_node_mask.astype(jnp.float32), (0, _NP - _N))

    feats128 = edge_feats.reshape(_E // 128, 2048)
    m = jnp.kron(jnp.eye(128, dtype=jnp.float32), W)
    eb = 200
    ebf = eb * 128
    rewards, exp_rewards, src, dst = pl.pallas_call(
        _encoder_body,
        grid=(_E // 128 // eb,),
        in_specs=[
            pl.BlockSpec((eb, 2048), lambda i: (i, 0)),
            pl.BlockSpec((2048, 128), lambda i: (0, 0)),
            pl.BlockSpec((1, 1), lambda i: (0, 0)),
            pl.BlockSpec((2, ebf), lambda i: (0, i)),
        ],
        out_specs=[
            pl.BlockSpec((ebf,), lambda i: (i,)),
            pl.BlockSpec((ebf,), lambda i: (i,)),
            pl.BlockSpec((ebf,), lambda i: (i,)),
            pl.BlockSpec((ebf,), lambda i: (i,)),
        ],
        out_shape=[
            jax.ShapeDtypeStruct((_E,), jnp.float32),
            jax.ShapeDtypeStruct((_E,), jnp.float32),
            jax.ShapeDtypeStruct((_E,), jnp.int32),
            jax.ShapeDtypeStruct((_E,), jnp.int32),
        ],
    )(feats128, m, b.reshape(1, 1), edge_index)

    mesh = plsc.VectorSubcoreMesh(
        core_axis_name="c", subcore_axis_name="s", num_cores=_NC)
    fp_iter = pl.kernel(
        _fp_iter_body,
        out_type=jax.ShapeDtypeStruct((_NC * _NP,), jnp.float32),
        mesh=mesh,
        compiler_params=pltpu.CompilerParams(needs_layout_passes=False),
        scratch_types=_SC_SCRATCH + [pltpu.VMEM_SHARED((_NP,), jnp.float32)],
    )
    fp_probs = pl.kernel(
        _probs_body,
        out_type=jax.ShapeDtypeStruct((_E,), jnp.float32),
        mesh=mesh,
        compiler_params=pltpu.CompilerParams(needs_layout_passes=False),
        scratch_types=_SC_SCRATCH,
    )

    def _step(p, _):
        return fp_iter(src, dst, exp_rewards, sinkf, p), None

    p0 = jnp.zeros((_NC * _NP,), jnp.float32)
    p, _ = lax.scan(_step, p0, None, length=_ITERS)

    edge_probs = fp_probs(src, dst, exp_rewards, p)

    values = pl.pallas_call(
        _log_body,
        out_shape=jax.ShapeDtypeStruct((784, 128), jnp.float32),
    )(p.reshape(_NC, 784, 128)).reshape(_NP)[:_N]

    return rewards, values, edge_probs
